# baseline (speedup 1.0000x reference)
import functools

import jax
import jax.numpy as jnp
from jax import lax
from jax.experimental import pallas as pl
from jax.experimental.pallas import tpu as pltpu
from jax.experimental.pallas import tpu_sc as plsc

_EMB = 64
_NEIGH = 30
_NP = 32
_NC, _NS = 2, 16
_NW = _NC * _NS
_CHUNK = 128
_GROUP = 2
_GR = _GROUP * _CHUNK


def _sc_gather(cat_table, rel_idx, ent_idx, n_rows):
    rows_per_w = n_rows // _NW
    chunks_per_w = rows_per_w // _CHUNK
    n_groups = chunks_per_w // _GROUP
    mesh = plsc.VectorSubcoreMesh(core_axis_name="c", subcore_axis_name="s")
    out_t = [jax.ShapeDtypeStruct((n_rows, 2 * _EMB), jnp.float32)] * 2

    @functools.partial(
        pl.kernel,
        out_type=out_t,
        mesh=mesh,
        scratch_types=[
            pltpu.VMEM((chunks_per_w, _CHUNK), jnp.int32),
            pltpu.VMEM((2 * _GR, 2 * _EMB), jnp.float32),
            pltpu.SemaphoreType.DMA,
            pltpu.SemaphoreType.DMA,
        ],
    )
    def gather_k(tab_hbm, rel_hbm, ent_hbm, o_rel, o_ent, idx_v, rows_v, gsem, wsem):
        wid = lax.axis_index("s") * _NC + lax.axis_index("c")
        base = wid * rows_per_w

        def fire(g, half):
            return [
                pltpu.async_copy(
                    tab_hbm.at[idx_v.at[g * _GROUP + j]],
                    rows_v.at[pl.ds(half + j * _CHUNK, _CHUNK)],
                    gsem,
                )
                for j in range(_GROUP)
            ]

        phase = wid * (n_groups // _NW)

        def rot(t):
            return lax.rem(phase + t, n_groups)

        for idx_hbm, out in ((rel_hbm, o_rel), (ent_hbm, o_ent)):
            pltpu.sync_copy(idx_hbm.at[wid], idx_v)
            for t in (0, 1):
                half = t * _GR
                g = rot(t)
                for h in fire(g, half):
                    h.wait()
                pltpu.async_copy(rows_v.at[pl.ds(half, _GR)],
                                 out.at[pl.ds(base + g * _GR, _GR)], wsem)

            def gbody(t, _):
                half = (t % 2) * _GR
                src = rows_v.at[pl.ds(half, _GR)]
                pltpu.make_async_copy(
                    src, out.at[pl.ds(base + rot(t - 2) * _GR, _GR)], wsem).wait()
                for h in fire(rot(t), half):
                    h.wait()
                pltpu.async_copy(src, out.at[pl.ds(base + rot(t) * _GR, _GR)], wsem)
                return 0

            lax.fori_loop(2, n_groups, gbody, 0)
            for t in (n_groups - 2, n_groups - 1):
                half = (t % 2) * _GR
                pltpu.make_async_copy(
                    rows_v.at[pl.ds(half, _GR)],
                    out.at[pl.ds(base + rot(t) * _GR, _GR)], wsem).wait()

    return gather_k(cat_table, rel_idx, ent_idx)


def _attn_body(catr_ref, cate_ref, p_ref, q_ref, bcat_ref, ucm_ref, ucv_ref,
               ubm_ref, ubv_ref, om_ref, ov_ref, *, bb):
    n = bb * _NP
    d2 = 2 * _EMB
    cate = cate_ref[...]
    h = jnp.tanh(
        jnp.dot(catr_ref[...], p_ref[...],
                preferred_element_type=jnp.float32)
        + jnp.dot(cate, q_ref[...], preferred_element_type=jnp.float32)
        + bcat_ref[...])
    h3 = h.reshape(bb, _NP, d2)
    lm = jnp.sum(h3 * ucm_ref[...].reshape(1, 1, d2), axis=-1) + ubm_ref[0, 0]
    lv = jnp.sum(h3 * ucv_ref[...].reshape(1, 1, d2), axis=-1) + ubv_ref[0, 0]
    pad = lax.broadcasted_iota(jnp.int32, (bb, _NP), 1) >= _NEIGH
    lm = jnp.where(pad, -1e30, lm)
    lv = jnp.where(pad, -1e30, lv)
    am = jnp.exp(lm - jnp.max(lm, axis=-1, keepdims=True))
    am = am / jnp.sum(am, axis=-1, keepdims=True)
    av = jnp.exp(lv - jnp.max(lv, axis=-1, keepdims=True))
    av = av / jnp.sum(av, axis=-1, keepdims=True)
    e3 = cate.reshape(bb, _NP, d2)[:, :, :_EMB]
    om_ref[...] = jnp.tanh(jnp.sum(am[:, :, None] * e3, axis=1))
    ov_ref[...] = jnp.tanh(jnp.sum(av[:, :, None] * e3, axis=1))


def _tc_attn(catr, cate, Wm, bm, um, ubm, Wv, bv, uv, ubv, b):
    bb = 256
    grid = (b // bb,)
    d2 = 2 * _EMB
    z = jnp.zeros((_EMB, _EMB), jnp.float32)
    p = jnp.concatenate(
        [jnp.concatenate([Wm[:_EMB], z], axis=1),
         jnp.concatenate([z, Wv[:_EMB]], axis=1)], axis=0)
    q = jnp.concatenate(
        [jnp.concatenate([Wm[_EMB:], z], axis=1),
         jnp.concatenate([z, Wv[_EMB:]], axis=1)], axis=0)
    bcat = jnp.concatenate([bm, bv]).reshape(1, d2)
    zu = jnp.zeros((_EMB,), jnp.float32)
    ucm = jnp.concatenate([um[:, 0], zu]).reshape(1, d2)
    ucv = jnp.concatenate([zu, uv[:, 0]]).reshape(1, d2)

    row_spec = pl.BlockSpec((bb * _NP, d2), lambda i: (i, 0))
    full = lambda shape: pl.BlockSpec(shape, lambda i: (0,) * len(shape))
    out_spec = pl.BlockSpec((bb, _EMB), lambda i: (i, 0))
    return pl.pallas_call(
        functools.partial(_attn_body, bb=bb),
        grid=grid,
        in_specs=[row_spec, row_spec,
                  full((d2, d2)), full((d2, d2)), full((1, d2)),
                  full((1, d2)), full((1, d2)),
                  full((1, 1)), full((1, 1))],
        out_specs=[out_spec, out_spec],
        out_shape=[jax.ShapeDtypeStruct((b, _EMB), jnp.float32)] * 2,
    )(catr, cate, p, q, bcat, ucm, ucv,
      ubm.reshape(1, 1), ubv.reshape(1, 1))


def kernel(connections, num_neighbors, emb_mean, emb_var,
           Wm, bm, um, ubm, Wv, bv, uv, ubv):
    b = connections.shape[0]
    cat_table = jnp.concatenate([emb_mean, emb_var], axis=1)
    padded = jnp.pad(connections[:, :, :2], ((0, 0), (0, _NP - _NEIGH), (0, 0)),
                     mode="edge")
    n_slabs = 8
    bs = b // n_slabs
    n_rows = bs * _NP
    chunks_per_w = n_rows // (_NW * _CHUNK)
    outs = []
    for s in range(n_slabs):
        pad_s = padded[s * bs:(s + 1) * bs]
        rel = pad_s[:, :, 0].reshape(_NW, chunks_per_w, _CHUNK)
        ent = pad_s[:, :, 1].reshape(_NW, chunks_per_w, _CHUNK)
        catr, cate = _sc_gather(cat_table, rel, ent, n_rows)
        outs.append(_tc_attn(catr, cate, Wm, bm, um, ubm, Wv, bv, uv, ubv, bs))
    return (jnp.concatenate([o[0] for o in outs]),
            jnp.concatenate([o[1] for o in outs]))


# speedup vs baseline: 1.0064x; 1.0064x over previous
import functools

import jax
import jax.numpy as jnp
from jax import lax
from jax.experimental import pallas as pl
from jax.experimental.pallas import tpu as pltpu
from jax.experimental.pallas import tpu_sc as plsc

_EMB = 64
_NEIGH = 30
_NP = 32
_NC, _NS = 2, 16
_NW = _NC * _NS
_CHUNK = 128
_GROUP = 2
_GR = _GROUP * _CHUNK


def _sc_gather(cat_table, rel_idx, ent_idx, n_rows):
    rows_per_w = n_rows // _NW
    chunks_per_w = rows_per_w // _CHUNK
    n_groups = chunks_per_w // _GROUP
    mesh = plsc.VectorSubcoreMesh(core_axis_name="c", subcore_axis_name="s")
    out_t = [jax.ShapeDtypeStruct((n_rows, 2 * _EMB), jnp.float32)] * 2

    @functools.partial(
        pl.kernel,
        out_type=out_t,
        mesh=mesh,
        scratch_types=[
            pltpu.VMEM((chunks_per_w, _CHUNK), jnp.int32),
            pltpu.VMEM((2 * _GR, 2 * _EMB), jnp.float32),
            pltpu.SemaphoreType.DMA,
            pltpu.SemaphoreType.DMA,
        ],
    )
    def gather_k(tab_hbm, rel_hbm, ent_hbm, o_rel, o_ent, idx_v, rows_v, gsem, wsem):
        wid = lax.axis_index("s") * _NC + lax.axis_index("c")
        base = wid * rows_per_w

        def fire(g, half):
            return [
                pltpu.async_copy(
                    tab_hbm.at[idx_v.at[g * _GROUP + j]],
                    rows_v.at[pl.ds(half + j * _CHUNK, _CHUNK)],
                    gsem,
                )
                for j in range(_GROUP)
            ]

        phase = wid * (n_groups // _NW)

        def rot(t):
            return lax.rem(phase + t, n_groups)

        for idx_hbm, out in ((rel_hbm, o_rel), (ent_hbm, o_ent)):
            pltpu.sync_copy(idx_hbm.at[wid], idx_v)
            for t in (0, 1):
                half = t * _GR
                g = rot(t)
                for h in fire(g, half):
                    h.wait()
                pltpu.async_copy(rows_v.at[pl.ds(half, _GR)],
                                 out.at[pl.ds(base + g * _GR, _GR)], wsem)

            def gbody(t, _):
                half = (t % 2) * _GR
                src = rows_v.at[pl.ds(half, _GR)]
                pltpu.make_async_copy(
                    src, out.at[pl.ds(base + rot(t - 2) * _GR, _GR)], wsem).wait()
                for h in fire(rot(t), half):
                    h.wait()
                pltpu.async_copy(src, out.at[pl.ds(base + rot(t) * _GR, _GR)], wsem)
                return 0

            lax.fori_loop(2, n_groups, gbody, 0)
            for t in (n_groups - 2, n_groups - 1):
                half = (t % 2) * _GR
                pltpu.make_async_copy(
                    rows_v.at[pl.ds(half, _GR)],
                    out.at[pl.ds(base + rot(t) * _GR, _GR)], wsem).wait()

    return gather_k(cat_table, rel_idx, ent_idx)


def _attn_body(catr_ref, cate_ref, p_ref, q_ref, bcat_ref, ucm_ref, ucv_ref,
               ubm_ref, ubv_ref, om_ref, ov_ref, *, bb):
    n = bb * _NP
    d2 = 2 * _EMB
    cate = cate_ref[...]
    h = jnp.tanh(
        jnp.dot(catr_ref[...], p_ref[...],
                preferred_element_type=jnp.float32)
        + jnp.dot(cate, q_ref[...], preferred_element_type=jnp.float32)
        + bcat_ref[...])
    h3 = h.reshape(bb, _NP, d2)
    lm = jnp.sum(h3 * ucm_ref[...].reshape(1, 1, d2), axis=-1) + ubm_ref[0, 0]
    lv = jnp.sum(h3 * ucv_ref[...].reshape(1, 1, d2), axis=-1) + ubv_ref[0, 0]
    pad = lax.broadcasted_iota(jnp.int32, (bb, _NP), 1) >= _NEIGH
    lm = jnp.where(pad, -1e30, lm)
    lv = jnp.where(pad, -1e30, lv)
    am = jnp.exp(lm - jnp.max(lm, axis=-1, keepdims=True))
    am = am / jnp.sum(am, axis=-1, keepdims=True)
    av = jnp.exp(lv - jnp.max(lv, axis=-1, keepdims=True))
    av = av / jnp.sum(av, axis=-1, keepdims=True)
    e3 = cate.reshape(bb, _NP, d2)[:, :, :_EMB]
    om_ref[...] = jnp.tanh(jnp.sum(am[:, :, None] * e3, axis=1))
    ov_ref[...] = jnp.tanh(jnp.sum(av[:, :, None] * e3, axis=1))


def _tc_attn(catr, cate, Wm, bm, um, ubm, Wv, bv, uv, ubv, b):
    bb = 256
    grid = (b // bb,)
    d2 = 2 * _EMB
    z = jnp.zeros((_EMB, _EMB), jnp.float32)
    p = jnp.concatenate(
        [jnp.concatenate([Wm[:_EMB], z], axis=1),
         jnp.concatenate([z, Wv[:_EMB]], axis=1)], axis=0)
    q = jnp.concatenate(
        [jnp.concatenate([Wm[_EMB:], z], axis=1),
         jnp.concatenate([z, Wv[_EMB:]], axis=1)], axis=0)
    bcat = jnp.concatenate([bm, bv]).reshape(1, d2)
    zu = jnp.zeros((_EMB,), jnp.float32)
    ucm = jnp.concatenate([um[:, 0], zu]).reshape(1, d2)
    ucv = jnp.concatenate([zu, uv[:, 0]]).reshape(1, d2)

    row_spec = pl.BlockSpec((bb * _NP, d2), lambda i: (i, 0))
    full = lambda shape: pl.BlockSpec(shape, lambda i: (0,) * len(shape))
    out_spec = pl.BlockSpec((bb, _EMB), lambda i: (i, 0))
    return pl.pallas_call(
        functools.partial(_attn_body, bb=bb),
        grid=grid,
        in_specs=[row_spec, row_spec,
                  full((d2, d2)), full((d2, d2)), full((1, d2)),
                  full((1, d2)), full((1, d2)),
                  full((1, 1)), full((1, 1))],
        out_specs=[out_spec, out_spec],
        out_shape=[jax.ShapeDtypeStruct((b, _EMB), jnp.float32)] * 2,
    )(catr, cate, p, q, bcat, ucm, ucv,
      ubm.reshape(1, 1), ubv.reshape(1, 1))


def kernel(connections, num_neighbors, emb_mean, emb_var,
           Wm, bm, um, ubm, Wv, bv, uv, ubv):
    b = connections.shape[0]
    cat_table = jnp.concatenate([emb_mean, emb_var], axis=1)
    padded = jnp.pad(connections[:, :, :2], ((0, 0), (0, _NP - _NEIGH), (0, 0)),
                     mode="edge")
    n_slabs = 4
    bs = b // n_slabs
    n_rows = bs * _NP
    chunks_per_w = n_rows // (_NW * _CHUNK)
    outs = []
    for s in range(n_slabs):
        pad_s = padded[s * bs:(s + 1) * bs]
        rel = pad_s[:, :, 0].reshape(_NW, chunks_per_w, _CHUNK)
        ent = pad_s[:, :, 1].reshape(_NW, chunks_per_w, _CHUNK)
        catr, cate = _sc_gather(cat_table, rel, ent, n_rows)
        outs.append(_tc_attn(catr, cate, Wm, bm, um, ubm, Wv, bv, uv, ubv, bs))
    return (jnp.concatenate([o[0] for o in outs]),
            jnp.concatenate([o[1] for o in outs]))
